# baseline (speedup 1.0000x reference)
import functools

import jax
import jax.numpy as jnp
from jax import lax
from jax.experimental import pallas as pl
from jax.experimental.pallas import tpu as pltpu
from jax.experimental.pallas import tpu_sc as plsc

_NC = 2
_NS = 16
_LANES = 16
_CHUNK = 80
_NB = 3
_G = 8
_NGB = 2


def _spmm(xr, meta, wr, n_nodes):
    _, dh = xr.shape
    n = n_nodes
    ng = meta.shape[1]
    nch = ng * _G
    rpt = (n // _NS) // 8 * 8
    rem = n - _NS * rpt

    mesh = plsc.VectorSubcoreMesh(
        core_axis_name="c", subcore_axis_name="s", num_cores=_NC, num_subcores=_NS
    )

    @functools.partial(
        pl.kernel,
        mesh=mesh,
        out_type=jax.ShapeDtypeStruct((_NC, n, dh), jnp.float32),
        scratch_types=[
            pltpu.VMEM((_NGB, _G, _CHUNK), jnp.int32),
            pltpu.VMEM((_NGB, _G, _CHUNK), jnp.int32),
            pltpu.VMEM((_NGB, _G, _CHUNK), jnp.float32),
            pltpu.VMEM((_NB, _CHUNK), jnp.int32),
            pltpu.VMEM((_NB, _CHUNK), jnp.int32),
            pltpu.VMEM((_NB, _CHUNK, dh), jnp.float32),
            pltpu.VMEM_SHARED((n, dh), jnp.float32),
            [pltpu.SemaphoreType.DMA] * _NGB,
            [pltpu.SemaphoreType.DMA] * _NB,
            [pltpu.SemaphoreType.DMA] * _NB,
        ],
    )
    def k(xr_hbm, meta_hbm, w_hbm, out_hbm, mgs, mgd, wgb, gb, db, rows, agg,
          sem_m, sem_g, sem_s):
        c = lax.axis_index("c")
        s = lax.axis_index("s")
        rbase = pl.multiple_of(s * rpt, 8)
        tbase = _NS * rpt

        def issue_meta(i, gsl):
            pltpu.async_copy(meta_hbm.at[s, i], mgs.at[gsl], sem_m[gsl])
            pltpu.async_copy(meta_hbm.at[_NS + s, i], mgd.at[gsl], sem_m[gsl])
            pltpu.async_copy(w_hbm.at[s, i], wgb.at[gsl], sem_m[gsl])

        def wait_meta(i, gsl):
            pltpu.make_async_copy(meta_hbm.at[s, i], mgs.at[gsl], sem_m[gsl]).wait()
            pltpu.make_async_copy(
                meta_hbm.at[_NS + s, i], mgd.at[gsl], sem_m[gsl]
            ).wait()
            pltpu.make_async_copy(w_hbm.at[s, i], wgb.at[gsl], sem_m[gsl]).wait()

        def issue_gather(b):
            pltpu.async_copy(xr_hbm.at[gb.at[b]], rows.at[b], sem_g[b])

        def wait_gather(b):
            pltpu.make_async_copy(xr_hbm.at[gb.at[b]], rows.at[b], sem_g[b]).wait()

        def issue_scatter(b):
            pltpu.async_copy(rows.at[b], agg.at[db.at[b]], sem_s[b], add=True)

        def wait_scatter(b):
            pltpu.make_async_copy(rows.at[b], agg.at[db.at[b]], sem_s[b]).wait()

        def ladder(fn, slot, nslots, *args):
            if isinstance(slot, int):
                fn(*args, slot) if args else fn(slot)
                return
            for q in range(nslots):
                @pl.when(slot == q)
                def _(q=q):
                    fn(*args, q) if args else fn(q)

        def prep_gather(b, gsl, kk):
            for v in range(_CHUNK // _LANES):
                sl = pl.ds(v * _LANES, _LANES)
                gb[b, sl] = mgs[gsl, kk, sl] * 2 + c
                db[b, sl] = mgd[gsl, kk, sl]
            ladder(issue_gather, b, _NB)

        def scale(b, gsl, kk):
            def grp(g, carry):
                wg = wgb[gsl, kk, pl.ds(g * _LANES, _LANES)]
                for r16 in range(_LANES):
                    r = g * _LANES + r16
                    wsc = wg[r16]
                    for v in range(dh // _LANES):
                        sl = pl.ds(v * _LANES, _LANES)
                        rows[b, r, sl] = rows[b, r, sl] * wsc
                return carry

            lax.fori_loop(0, _CHUNK // _LANES, grp, None)

        def zrow(r, carry):
            for v in range(dh // _LANES):
                rows[0, r, pl.ds(v * _LANES, _LANES)] = jnp.zeros(
                    (_LANES,), jnp.float32
                )
            return carry

        issue_meta(0, 0)
        lax.fori_loop(0, _CHUNK, zrow, None)
        nz_full = rpt // _CHUNK
        for kz in range(nz_full):
            pltpu.sync_copy(rows.at[0], agg.at[pl.ds(rbase + kz * _CHUNK, _CHUNK)])
        zrem = rpt - nz_full * _CHUNK
        if zrem:
            pltpu.sync_copy(
                rows.at[0, pl.ds(0, zrem)],
                agg.at[pl.ds(rbase + nz_full * _CHUNK, zrem)],
            )
        if rem:
            @pl.when(s == _NS - 1)
            def _zero_tail():
                pltpu.sync_copy(rows.at[0, pl.ds(0, rem)], agg.at[pl.ds(tbase, rem)])
        plsc.subcore_barrier()

        wait_meta(0, 0)
        prep_gather(0, 0, 0)

        def body(j, carry):
            b = lax.rem(j, _NB)
            bn = lax.rem(j + 1, _NB)
            @pl.when(j >= 2)
            def _():
                ladder(wait_scatter, bn, _NB)

            @pl.when(j + 1 < nch)
            def _():
                jn = j + 1
                @pl.when(lax.rem(jn, _G) == 0)
                def _():
                    ladder(wait_meta, lax.rem(jn // _G, _NGB), _NGB, jn // _G)
                prep_gather(bn, lax.rem(jn // _G, _NGB), lax.rem(jn, _G))

            @pl.when((lax.rem(j, _G) == 0) & (j < (ng - 1) * _G))
            def _():
                i = j // _G + 1
                ladder(issue_meta, lax.rem(i, _NGB), _NGB, i)

            ladder(wait_gather, b, _NB)
            scale(b, lax.rem(j // _G, _NGB), lax.rem(j, _G))
            ladder(issue_scatter, b, _NB)
            return carry

        lax.fori_loop(0, nch, body, None)
        wait_scatter((nch - 2) % _NB)
        wait_scatter((nch - 1) % _NB)
        plsc.subcore_barrier()

        pltpu.sync_copy(agg.at[pl.ds(rbase, rpt)], out_hbm.at[c, pl.ds(rbase, rpt)])
        if rem:
            @pl.when(s == _NS - 1)
            def _write_tail():
                pltpu.sync_copy(agg.at[pl.ds(tbase, rem)], out_hbm.at[c, pl.ds(tbase, rem)])

    return k(xr, meta, wr)


def _dense_relu(agg, W):
    _, n, dh = agg.shape
    d_out = W.shape[1]
    bm = 1000

    def body(a_ref, w_ref, o_ref):
        a = a_ref[...]
        w = w_ref[...]
        y = jnp.dot(a[0], w[:dh], preferred_element_type=jnp.float32)
        y = y + jnp.dot(a[1], w[dh:], preferred_element_type=jnp.float32)
        o_ref[...] = jnp.maximum(y, 0.0)

    return pl.pallas_call(
        body,
        grid=(n // bm,),
        in_specs=[
            pl.BlockSpec((2, bm, dh), lambda i: (0, i, 0)),
            pl.BlockSpec(W.shape, lambda i: (0, 0)),
        ],
        out_specs=pl.BlockSpec((bm, d_out), lambda i: (i, 0)),
        out_shape=jax.ShapeDtypeStruct((n, d_out), jnp.float32),
    )(agg, W)


def kernel(x, edge_index, edge_weight, W):
    n, d = x.shape
    e = edge_weight.shape[0]
    dh = d // 2
    xr = x.reshape(2 * n, dh)
    ept = e // _NS
    gsz = _G * _CHUNK
    ng = -(-ept // gsz)
    eptp = ng * gsz
    ei = edge_index.astype(jnp.int32).reshape(2, _NS, ept)
    ei = jnp.concatenate(
        [ei, jnp.zeros((2, _NS, eptp - ept), jnp.int32)], axis=2
    )
    meta = ei.reshape(2 * _NS, ng, _G, _CHUNK)
    wv = edge_weight.reshape(_NS, ept)
    wv = jnp.concatenate(
        [wv, jnp.zeros((_NS, eptp - ept), jnp.float32)], axis=1
    )
    wr = wv.reshape(_NS, ng, _G, _CHUNK)
    agg = _spmm(xr, meta, wr, n)
    return _dense_relu(agg, W)


# speedup vs baseline: 1.0016x; 1.0016x over previous
import functools

import jax
import jax.numpy as jnp
from jax import lax
from jax.experimental import pallas as pl
from jax.experimental.pallas import tpu as pltpu
from jax.experimental.pallas import tpu_sc as plsc

_NC = 2
_NS = 16
_LANES = 16
_CHUNK = 80
_NB = 4
_G = 8
_NGB = 2


def _spmm(xr, meta, wr, n_nodes):
    _, dh = xr.shape
    n = n_nodes
    ng = meta.shape[1]
    nch = ng * _G
    rpt = (n // _NS) // 8 * 8
    rem = n - _NS * rpt

    mesh = plsc.VectorSubcoreMesh(
        core_axis_name="c", subcore_axis_name="s", num_cores=_NC, num_subcores=_NS
    )

    @functools.partial(
        pl.kernel,
        mesh=mesh,
        out_type=jax.ShapeDtypeStruct((_NC, n, dh), jnp.float32),
        scratch_types=[
            pltpu.VMEM((_NGB, _G, _CHUNK), jnp.int32),
            pltpu.VMEM((_NGB, _G, _CHUNK), jnp.int32),
            pltpu.VMEM((_NGB, _G, _CHUNK), jnp.float32),
            pltpu.VMEM((_NB, _CHUNK), jnp.int32),
            pltpu.VMEM((_NB, _CHUNK), jnp.int32),
            pltpu.VMEM((_NB, _CHUNK, dh), jnp.float32),
            pltpu.VMEM_SHARED((n, dh), jnp.float32),
            [pltpu.SemaphoreType.DMA] * _NGB,
            [pltpu.SemaphoreType.DMA] * _NB,
            [pltpu.SemaphoreType.DMA] * _NB,
        ],
    )
    def k(xr_hbm, meta_hbm, w_hbm, out_hbm, mgs, mgd, wgb, gb, db, rows, agg,
          sem_m, sem_g, sem_s):
        c = lax.axis_index("c")
        s = lax.axis_index("s")
        rbase = pl.multiple_of(s * rpt, 8)
        tbase = _NS * rpt

        def issue_meta(i, gsl):
            pltpu.async_copy(meta_hbm.at[s, i], mgs.at[gsl], sem_m[gsl])
            pltpu.async_copy(meta_hbm.at[_NS + s, i], mgd.at[gsl], sem_m[gsl])
            pltpu.async_copy(w_hbm.at[s, i], wgb.at[gsl], sem_m[gsl])

        def wait_meta(i, gsl):
            pltpu.make_async_copy(meta_hbm.at[s, i], mgs.at[gsl], sem_m[gsl]).wait()
            pltpu.make_async_copy(
                meta_hbm.at[_NS + s, i], mgd.at[gsl], sem_m[gsl]
            ).wait()
            pltpu.make_async_copy(w_hbm.at[s, i], wgb.at[gsl], sem_m[gsl]).wait()

        def issue_gather(b):
            pltpu.async_copy(xr_hbm.at[gb.at[b]], rows.at[b], sem_g[b])

        def wait_gather(b):
            pltpu.make_async_copy(xr_hbm.at[gb.at[b]], rows.at[b], sem_g[b]).wait()

        def issue_scatter(b):
            pltpu.async_copy(rows.at[b], agg.at[db.at[b]], sem_s[b], add=True)

        def wait_scatter(b):
            pltpu.make_async_copy(rows.at[b], agg.at[db.at[b]], sem_s[b]).wait()

        def ladder(fn, slot, nslots, *args):
            if isinstance(slot, int):
                fn(*args, slot) if args else fn(slot)
                return
            for q in range(nslots):
                @pl.when(slot == q)
                def _(q=q):
                    fn(*args, q) if args else fn(q)

        def prep_gather(b, gsl, kk):
            for v in range(_CHUNK // _LANES):
                sl = pl.ds(v * _LANES, _LANES)
                gb[b, sl] = mgs[gsl, kk, sl] * 2 + c
                db[b, sl] = mgd[gsl, kk, sl]
            ladder(issue_gather, b, _NB)

        def scale(b, gsl, kk):
            def grp(g, carry):
                wg = wgb[gsl, kk, pl.ds(g * _LANES, _LANES)]
                for r16 in range(_LANES):
                    r = g * _LANES + r16
                    wsc = wg[r16]
                    for v in range(dh // _LANES):
                        sl = pl.ds(v * _LANES, _LANES)
                        rows[b, r, sl] = rows[b, r, sl] * wsc
                return carry

            lax.fori_loop(0, _CHUNK // _LANES, grp, None)

        def zrow(r, carry):
            for v in range(dh // _LANES):
                rows[0, r, pl.ds(v * _LANES, _LANES)] = jnp.zeros(
                    (_LANES,), jnp.float32
                )
            return carry

        issue_meta(0, 0)
        lax.fori_loop(0, _CHUNK, zrow, None)
        nz_full = rpt // _CHUNK
        for kz in range(nz_full):
            pltpu.sync_copy(rows.at[0], agg.at[pl.ds(rbase + kz * _CHUNK, _CHUNK)])
        zrem = rpt - nz_full * _CHUNK
        if zrem:
            pltpu.sync_copy(
                rows.at[0, pl.ds(0, zrem)],
                agg.at[pl.ds(rbase + nz_full * _CHUNK, zrem)],
            )
        if rem:
            @pl.when(s == _NS - 1)
            def _zero_tail():
                pltpu.sync_copy(rows.at[0, pl.ds(0, rem)], agg.at[pl.ds(tbase, rem)])
        plsc.subcore_barrier()

        wait_meta(0, 0)
        prep_gather(0, 0, 0)
        prep_gather(1, 0, 1)

        def body(j, carry):
            b = lax.rem(j, _NB)
            bp = lax.rem(j + 2, _NB)
            @pl.when(j >= 2)
            def _():
                ladder(wait_scatter, bp, _NB)

            @pl.when(j + 2 < nch)
            def _():
                jn = j + 2
                @pl.when(lax.rem(jn, _G) == 0)
                def _():
                    ladder(wait_meta, lax.rem(jn // _G, _NGB), _NGB, jn // _G)
                prep_gather(bp, lax.rem(jn // _G, _NGB), lax.rem(jn, _G))

            @pl.when((lax.rem(j, _G) == 0) & (j < (ng - 1) * _G))
            def _():
                i = j // _G + 1
                ladder(issue_meta, lax.rem(i, _NGB), _NGB, i)

            ladder(wait_gather, b, _NB)
            scale(b, lax.rem(j // _G, _NGB), lax.rem(j, _G))
            ladder(issue_scatter, b, _NB)
            return carry

        lax.fori_loop(0, nch, body, None)
        wait_scatter((nch - 2) % _NB)
        wait_scatter((nch - 1) % _NB)
        plsc.subcore_barrier()

        pltpu.sync_copy(agg.at[pl.ds(rbase, rpt)], out_hbm.at[c, pl.ds(rbase, rpt)])
        if rem:
            @pl.when(s == _NS - 1)
            def _write_tail():
                pltpu.sync_copy(agg.at[pl.ds(tbase, rem)], out_hbm.at[c, pl.ds(tbase, rem)])

    return k(xr, meta, wr)


def _dense_relu(agg, W):
    _, n, dh = agg.shape
    d_out = W.shape[1]
    bm = 1000

    def body(a_ref, w_ref, o_ref):
        a = a_ref[...]
        w = w_ref[...]
        y = jnp.dot(a[0], w[:dh], preferred_element_type=jnp.float32)
        y = y + jnp.dot(a[1], w[dh:], preferred_element_type=jnp.float32)
        o_ref[...] = jnp.maximum(y, 0.0)

    return pl.pallas_call(
        body,
        grid=(n // bm,),
        in_specs=[
            pl.BlockSpec((2, bm, dh), lambda i: (0, i, 0)),
            pl.BlockSpec(W.shape, lambda i: (0, 0)),
        ],
        out_specs=pl.BlockSpec((bm, d_out), lambda i: (i, 0)),
        out_shape=jax.ShapeDtypeStruct((n, d_out), jnp.float32),
    )(agg, W)


def kernel(x, edge_index, edge_weight, W):
    n, d = x.shape
    e = edge_weight.shape[0]
    dh = d // 2
    xr = x.reshape(2 * n, dh)
    ept = e // _NS
    gsz = _G * _CHUNK
    ng = -(-ept // gsz)
    eptp = ng * gsz
    ei = edge_index.astype(jnp.int32).reshape(2, _NS, ept)
    ei = jnp.concatenate(
        [ei, jnp.zeros((2, _NS, eptp - ept), jnp.int32)], axis=2
    )
    meta = ei.reshape(2 * _NS, ng, _G, _CHUNK)
    wv = edge_weight.reshape(_NS, ept)
    wv = jnp.concatenate(
        [wv, jnp.zeros((_NS, eptp - ept), jnp.float32)], axis=1
    )
    wr = wv.reshape(_NS, ng, _G, _CHUNK)
    agg = _spmm(xr, meta, wr, n)
    return _dense_relu(agg, W)


# speedup vs baseline: 1.8657x; 1.8626x over previous
import functools

import jax
import jax.numpy as jnp
from jax import lax
from jax.experimental import pallas as pl
from jax.experimental.pallas import tpu as pltpu
from jax.experimental.pallas import tpu_sc as plsc

_NC = 2
_NS = 16
_LANES = 16
_CHUNK = 80
_NB = 4
_G = 8
_NGB = 2


def _spmm(xr, meta, wr, n_nodes):
    _, dh = xr.shape
    n = n_nodes
    ng = meta.shape[1]
    nch = ng * _G
    rpt = (n // _NS) // 8 * 8
    rem = n - _NS * rpt

    mesh = plsc.VectorSubcoreMesh(
        core_axis_name="c", subcore_axis_name="s", num_cores=_NC, num_subcores=_NS
    )

    @functools.partial(
        pl.kernel,
        mesh=mesh,
        out_type=jax.ShapeDtypeStruct((_NC, n, dh), jnp.float32),
        scratch_types=[
            pltpu.VMEM((_NGB, _G, _CHUNK), jnp.int32),
            pltpu.VMEM((_NGB, _G, _CHUNK), jnp.int32),
            pltpu.VMEM((_NGB, _G, _CHUNK), jnp.float32),
            pltpu.VMEM((_NB, _CHUNK), jnp.int32),
            pltpu.VMEM((_NB, _CHUNK), jnp.int32),
            pltpu.VMEM((_NB, _CHUNK, dh), jnp.float32),
            pltpu.VMEM_SHARED((n, dh), jnp.float32),
            [pltpu.SemaphoreType.DMA] * _NGB,
            [pltpu.SemaphoreType.DMA] * _NB,
            [pltpu.SemaphoreType.DMA] * _NB,
        ],
    )
    def k(xr_hbm, meta_hbm, w_hbm, out_hbm, mgs, mgd, wgb, gb, db, rows, agg,
          sem_m, sem_g, sem_s):
        c = lax.axis_index("c")
        s = lax.axis_index("s")
        rbase = pl.multiple_of(s * rpt, 8)
        tbase = _NS * rpt

        def issue_meta(i, gsl):
            pltpu.async_copy(meta_hbm.at[s, i], mgs.at[gsl], sem_m[gsl])
            pltpu.async_copy(meta_hbm.at[_NS + s, i], mgd.at[gsl], sem_m[gsl])
            pltpu.async_copy(w_hbm.at[s, i], wgb.at[gsl], sem_m[gsl])

        def wait_meta(i, gsl):
            pltpu.make_async_copy(meta_hbm.at[s, i], mgs.at[gsl], sem_m[gsl]).wait()
            pltpu.make_async_copy(
                meta_hbm.at[_NS + s, i], mgd.at[gsl], sem_m[gsl]
            ).wait()
            pltpu.make_async_copy(w_hbm.at[s, i], wgb.at[gsl], sem_m[gsl]).wait()

        def issue_gather(b):
            pltpu.async_copy(xr_hbm.at[gb.at[b]], rows.at[b], sem_g[b])

        def wait_gather(b):
            pltpu.make_async_copy(xr_hbm.at[gb.at[b]], rows.at[b], sem_g[b]).wait()

        def issue_scatter(b):
            pltpu.async_copy(rows.at[b], agg.at[db.at[b]], sem_s[b], add=True)

        def wait_scatter(b):
            pltpu.make_async_copy(rows.at[b], agg.at[db.at[b]], sem_s[b]).wait()

        def ladder(fn, slot, nslots, *args):
            if isinstance(slot, int):
                fn(*args, slot) if args else fn(slot)
                return
            for q in range(nslots):
                @pl.when(slot == q)
                def _(q=q):
                    fn(*args, q) if args else fn(q)

        def prep_gather(b, gsl, kk):
            for v in range(_CHUNK // _LANES):
                sl = pl.ds(v * _LANES, _LANES)
                gb[b, sl] = mgs[gsl, kk, sl] * 2 + c
                db[b, sl] = mgd[gsl, kk, sl]
            issue_gather(b)

        def scale(b, gsl, kk):
            def grp(g, carry):
                wg = wgb[gsl, kk, pl.ds(g * _LANES, _LANES)]
                for r16 in range(_LANES):
                    r = g * _LANES + r16
                    wsc = wg[r16]
                    for v in range(dh // _LANES):
                        sl = pl.ds(v * _LANES, _LANES)
                        rows[b, r, sl] = rows[b, r, sl] * wsc
                return carry

            lax.fori_loop(0, _CHUNK // _LANES, grp, None)

        def zrow(r, carry):
            for v in range(dh // _LANES):
                rows[0, r, pl.ds(v * _LANES, _LANES)] = jnp.zeros(
                    (_LANES,), jnp.float32
                )
            return carry

        issue_meta(0, 0)
        lax.fori_loop(0, _CHUNK, zrow, None)
        nz_full = rpt // _CHUNK
        for kz in range(nz_full):
            pltpu.sync_copy(rows.at[0], agg.at[pl.ds(rbase + kz * _CHUNK, _CHUNK)])
        zrem = rpt - nz_full * _CHUNK
        if zrem:
            pltpu.sync_copy(
                rows.at[0, pl.ds(0, zrem)],
                agg.at[pl.ds(rbase + nz_full * _CHUNK, zrem)],
            )
        if rem:
            @pl.when(s == _NS - 1)
            def _zero_tail():
                pltpu.sync_copy(rows.at[0, pl.ds(0, rem)], agg.at[pl.ds(tbase, rem)])
        plsc.subcore_barrier()

        wait_meta(0, 0)
        prep_gather(0, 0, 0)
        prep_gather(1, 0, 1)
        plsc.subcore_barrier()

        def emit_chunk(kk, isl, prep, ws=True):
            b = kk % _NB
            if ws:
                wait_scatter((kk + 2) % _NB)
            if prep is not None:
                prep_gather(*prep)
            wait_gather(b)
            scale(b, isl, kk)
            issue_scatter(b)

        issue_meta(1, 1)
        for kk in range(_G):
            pj = kk + 2
            if pj == _G:
                wait_meta(1, 1)
            prep = (pj % _NB, 0 if pj < _G else 1, pj if pj < _G else pj - _G)
            emit_chunk(kk, 0, prep, ws=(kk >= 2))

        def gbody(i, carry):
            isl = lax.rem(i, _NGB)
            nsl = lax.rem(i + 1, _NGB)
            ladder(issue_meta, nsl, _NGB, i + 1)
            for kk in range(_G):
                pj = kk + 2
                if pj == _G:
                    ladder(wait_meta, nsl, _NGB, i + 1)
                pgsl = isl if pj < _G else nsl
                pkk = pj if pj < _G else pj - _G
                emit_chunk(kk, isl, (pj % _NB, pgsl, pkk))
            return carry

        if ng > 2:
            lax.fori_loop(1, ng - 1, gbody, None)

        lsl = (ng - 1) % _NGB
        for kk in range(_G):
            pj = kk + 2
            prep = (pj % _NB, lsl, pj) if pj < _G else None
            emit_chunk(kk, lsl, prep)
        wait_scatter((_G - 2) % _NB)
        wait_scatter((_G - 1) % _NB)
        plsc.subcore_barrier()

        pltpu.sync_copy(agg.at[pl.ds(rbase, rpt)], out_hbm.at[c, pl.ds(rbase, rpt)])
        if rem:
            @pl.when(s == _NS - 1)
            def _write_tail():
                pltpu.sync_copy(agg.at[pl.ds(tbase, rem)], out_hbm.at[c, pl.ds(tbase, rem)])

    return k(xr, meta, wr)


def _dense_relu(agg, W):
    _, n, dh = agg.shape
    d_out = W.shape[1]
    bm = 1000

    def body(a_ref, w_ref, o_ref):
        a = a_ref[...]
        w = w_ref[...]
        y = jnp.dot(a[0], w[:dh], preferred_element_type=jnp.float32)
        y = y + jnp.dot(a[1], w[dh:], preferred_element_type=jnp.float32)
        o_ref[...] = jnp.maximum(y, 0.0)

    return pl.pallas_call(
        body,
        grid=(n // bm,),
        in_specs=[
            pl.BlockSpec((2, bm, dh), lambda i: (0, i, 0)),
            pl.BlockSpec(W.shape, lambda i: (0, 0)),
        ],
        out_specs=pl.BlockSpec((bm, d_out), lambda i: (i, 0)),
        out_shape=jax.ShapeDtypeStruct((n, d_out), jnp.float32),
    )(agg, W)


def kernel(x, edge_index, edge_weight, W):
    n, d = x.shape
    e = edge_weight.shape[0]
    dh = d // 2
    xr = x.reshape(2 * n, dh)
    ept = e // _NS
    gsz = _G * _CHUNK
    ng = -(-ept // gsz)
    eptp = ng * gsz
    ei = edge_index.astype(jnp.int32).reshape(2, _NS, ept)
    ei = jnp.concatenate(
        [ei, jnp.zeros((2, _NS, eptp - ept), jnp.int32)], axis=2
    )
    meta = ei.reshape(2 * _NS, ng, _G, _CHUNK)
    wv = edge_weight.reshape(_NS, ept)
    wv = jnp.concatenate(
        [wv, jnp.zeros((_NS, eptp - ept), jnp.float32)], axis=1
    )
    wr = wv.reshape(_NS, ng, _G, _CHUNK)
    agg = _spmm(xr, meta, wr, n)
    return _dense_relu(agg, W)
